# baseline (speedup 1.0000x reference)
import jax
import jax.numpy as jnp
import numpy as np
from jax import lax
from jax.experimental import pallas as pl
from jax.experimental.pallas import tpu as pltpu
from jax.experimental.pallas import tpu_sc as plsc

N = 10000
E = 160000
H = 256
HALF = 128
G = 64
CUTOFF = 10.0


BN = 2000


def _xf_body(x_ref, w_ref, out_ref):
    r = jnp.dot(x_ref[...], w_ref[...], preferred_element_type=jnp.float32)
    out_ref[0] = r[:, :HALF]
    out_ref[1] = r[:, HALF:]


def _xf(x, lin1_t):
    return pl.pallas_call(
        _xf_body,
        grid=(N // BN,),
        in_specs=[
            pl.BlockSpec((BN, H), lambda i: (i, 0)),
            pl.BlockSpec((H, H), lambda i: (0, 0)),
        ],
        out_specs=pl.BlockSpec((2, BN, HALF), lambda i: (0, i, 0)),
        out_shape=jax.ShapeDtypeStruct((2, N, HALF), jnp.float32),
    )(x, lin1_t)


BE = 1280


def _ssp(v):
    return jax.nn.softplus(v) - jnp.log(2.0)


def _filt_body(eat_ref, el_ref, w1_ref, b1_ref, w2_ref, b2_ref, out_ref):
    ht = jnp.dot(w1_ref[...], eat_ref[...], preferred_element_type=jnp.float32)
    ht = _ssp(ht + b1_ref[...])
    wt = jnp.dot(w2_ref[...], ht, preferred_element_type=jnp.float32) + b2_ref[...]
    el = el_ref[...]
    c = 0.5 * (jnp.cos(el * (jnp.pi / CUTOFF)) + 1.0)
    c = c * (el <= CUTOFF).astype(jnp.float32) * (el >= 0.0).astype(jnp.float32)
    wt = wt * c
    w = wt.T.astype(jnp.bfloat16)
    out_ref[0] = w[:, :HALF]
    out_ref[1] = w[:, HALF:]


def _filt(edge_attr_t, el2d, w1, b1, w2, b2):
    return pl.pallas_call(
        _filt_body,
        grid=(E // BE,),
        in_specs=[
            pl.BlockSpec((G, BE), lambda i: (0, i)),
            pl.BlockSpec((1, BE), lambda i: (0, i)),
            pl.BlockSpec((H, G), lambda i: (0, 0)),
            pl.BlockSpec((H, 1), lambda i: (0, 0)),
            pl.BlockSpec((H, H), lambda i: (0, 0)),
            pl.BlockSpec((H, 1), lambda i: (0, 0)),
        ],
        out_specs=pl.BlockSpec((2, BE, HALF), lambda i: (0, i, 0)),
        out_shape=jax.ShapeDtypeStruct((2, E, HALF), jnp.bfloat16),
    )(edge_attr_t, el2d, w1, b1, w2, b2)


B = 80
EPS = E // 16
NCHUNK = EPS // B
SPS = 632
NP = 16 * SPS
PK = HALF // 2


def _sc_body(xf2, w2, src2, dst, agg2, sidxs, didx0, didx1, xr0, xr1, wr,
             accsh, semg0, semg1, semw, semd0, semd1, semsc0, semsc1):
    c = lax.axis_index("c")
    s = lax.axis_index("s")

    pltpu.sync_copy(src2.at[pl.ds(c * E + s * EPS, EPS)], sidxs)

    zero16 = jnp.zeros((16,), jnp.float32)
    for i in range(B):
        for j in range(HALF // 16):
            xr0[i, pl.ds(j * 16, 16)] = zero16
    full, rem = divmod(SPS, B)
    for t in range(full):
        pltpu.sync_copy(xr0, accsh.at[pl.ds(s * SPS + t * B, B)])
    if rem:
        pltpu.sync_copy(xr0.at[pl.ds(0, rem)],
                        accsh.at[pl.ds(s * SPS + full * B, rem)])
    plsc.subcore_barrier()

    ebase = s * EPS
    gbase = c * E + ebase

    pltpu.async_copy(dst.at[pl.ds(ebase, B)], didx0, semd0)
    pltpu.async_copy(w2.at[pl.ds(gbase, B)], wr, semw)
    pltpu.async_copy(xf2.at[sidxs.at[pl.ds(0, B)]], xr0, semg0)

    himask = jnp.full((16,), -65536, jnp.int32)

    himask = jnp.full((16,), -65536, jnp.int32)
    shamt = jnp.full((16,), 16, jnp.int32)

    def chunk(k, carry):
        def run(xr, semg, di, semd, semsc, xr_o, semg_o, di_o, semd_o,
                semsc_o):
            @pl.when(k >= 1)
            def _drain_sc():
                pltpu.make_async_copy(xr_o, accsh.at[di_o], semsc_o).wait()

            @pl.when(k + 1 < NCHUNK)
            def _issue():
                pltpu.async_copy(
                    xf2.at[sidxs.at[pl.ds((k + 1) * B, B)]], xr_o, semg_o)
                pltpu.async_copy(
                    dst.at[pl.ds(ebase + (k + 1) * B, B)], di_o, semd_o)

            pltpu.make_async_copy(xf2.at[pl.ds(0, B)], xr, semg).wait()
            pltpu.make_async_copy(w2.at[pl.ds(0, B)], wr, semw).wait()

            for i in range(B):
                for j in range(HALF // 32):
                    wi = wr[i, pl.ds(j * 16, 16)]
                    w_lo = lax.bitcast_convert_type(lax.shift_left(wi, shamt), jnp.float32)
                    w_hi = lax.bitcast_convert_type(wi & himask, jnp.float32)
                    sl_lo = pl.ds(j * 32, 16)
                    sl_hi = pl.ds(j * 32 + 16, 16)
                    xr[i, sl_lo] = xr[i, sl_lo] * w_lo
                    xr[i, sl_hi] = xr[i, sl_hi] * w_hi

            @pl.when(k + 1 < NCHUNK)
            def _issue_w():
                pltpu.async_copy(
                    w2.at[pl.ds(gbase + (k + 1) * B, B)], wr, semw)

            pltpu.make_async_copy(dst.at[pl.ds(0, B)], di, semd).wait()
            pltpu.async_copy(xr, accsh.at[di], semsc, add=True)

        @pl.when(k % 2 == 0)
        def _even():
            run(xr0, semg0, didx0, semd0, semsc0,
                xr1, semg1, didx1, semd1, semsc1)

        @pl.when(k % 2 == 1)
        def _odd():
            run(xr1, semg1, didx1, semd1, semsc1,
                xr0, semg0, didx0, semd0, semsc0)

        return carry

    lax.fori_loop(0, NCHUNK, chunk, 0)
    pltpu.make_async_copy(xr0, accsh.at[didx0], semsc0).wait()
    plsc.subcore_barrier()

    pltpu.sync_copy(accsh.at[pl.ds(s * SPS, SPS)],
                    agg2.at[pl.ds(c * NP + s * SPS, SPS)])


def _sc_aggregate(xf2, w2, src2, dst):
    mesh = plsc.VectorSubcoreMesh(core_axis_name="c", subcore_axis_name="s",
                                  num_cores=2, num_subcores=16)
    return pl.kernel(
        _sc_body,
        out_type=jax.ShapeDtypeStruct((2 * NP, HALF), jnp.float32),
        mesh=mesh,
        scratch_types=[
            pltpu.VMEM((EPS,), jnp.int32),
            pltpu.VMEM((B,), jnp.int32),
            pltpu.VMEM((B,), jnp.int32),
            pltpu.VMEM((B, HALF), jnp.float32),
            pltpu.VMEM((B, HALF), jnp.float32),
            pltpu.VMEM((B, PK), jnp.int32),
            pltpu.VMEM_SHARED((NP, HALF), jnp.float32),
            pltpu.SemaphoreType.DMA,
            pltpu.SemaphoreType.DMA,
            pltpu.SemaphoreType.DMA,
            pltpu.SemaphoreType.DMA,
            pltpu.SemaphoreType.DMA,
            pltpu.SemaphoreType.DMA,
            pltpu.SemaphoreType.DMA,
        ],
    )(xf2, w2, src2, dst)


def _out_body(agg_ref, w2_ref, b2_ref, w_ref, b_ref, out_ref):
    a0 = agg_ref[0].astype(jnp.float32)
    a1 = agg_ref[1].astype(jnp.float32)
    r = (jnp.dot(a0, w2_ref[:HALF], preferred_element_type=jnp.float32)
         + jnp.dot(a1, w2_ref[HALF:], preferred_element_type=jnp.float32)
         + b2_ref[...])
    r = _ssp(r)
    out_ref[...] = jnp.dot(r, w_ref[...], preferred_element_type=jnp.float32) + b_ref[...]


def _out_mlp(agg2, lin2_t, lin2_b, lin_t, lin_b):
    return pl.pallas_call(
        _out_body,
        grid=(N // BN,),
        in_specs=[
            pl.BlockSpec((2, BN, HALF), lambda i: (0, i, 0)),
            pl.BlockSpec((H, H), lambda i: (0, 0)),
            pl.BlockSpec((1, H), lambda i: (0, 0)),
            pl.BlockSpec((H, H), lambda i: (0, 0)),
            pl.BlockSpec((1, H), lambda i: (0, 0)),
        ],
        out_specs=pl.BlockSpec((BN, H), lambda i: (i, 0)),
        out_shape=jax.ShapeDtypeStruct((N, H), jnp.float32),
    )(agg2, lin2_t, lin2_b, lin_t, lin_b)


_PERM = np.array(
    [32 * (c // 32) + 2 * (c % 32) if (c % 32) < 16
     else 32 * (c // 32) + 2 * ((c % 32) - 16) + 1
     for c in range(HALF)], dtype=np.int32)
_PERM_FULL = np.concatenate([_PERM, HALF + _PERM])


def kernel(x, edge_index, edge_length, edge_attr, lin1_w, mlp_w1, mlp_b1,
           mlp_w2, mlp_b2, lin2_w, lin2_b, lin_w, lin_b):
    src = edge_index[0].astype(jnp.int32)
    dst = edge_index[1].astype(jnp.int32)
    src2 = jnp.concatenate([src, src + N])

    xf2 = _xf(x, lin1_w[_PERM_FULL].T).reshape(2 * N, HALF)
    w2 = _filt(edge_attr.T, edge_length.reshape(1, E),
               mlp_w1, mlp_b1.reshape(H, 1), mlp_w2,
               mlp_b2.reshape(H, 1)).reshape(2 * E, PK, 2)
    w2i = jax.lax.bitcast_convert_type(w2, jnp.int32)
    agg2 = _sc_aggregate(xf2, w2i, src2, dst).reshape(2, NP, HALF)
    out = _out_mlp(agg2, lin2_w.T[_PERM_FULL], lin2_b.reshape(1, H),
                   lin_w.T, lin_b.reshape(1, H))
    return out


# speedup vs baseline: 3.5984x; 3.5984x over previous
import jax
import jax.numpy as jnp
import numpy as np
from jax import lax
from jax.experimental import pallas as pl
from jax.experimental.pallas import tpu as pltpu
from jax.experimental.pallas import tpu_sc as plsc

N = 10000
E = 160000
H = 256
HALF = 128
G = 64
CUTOFF = 10.0


BN = 2000


def _xf_body(x_ref, w_ref, out_ref):
    r = jnp.dot(x_ref[...], w_ref[...], preferred_element_type=jnp.float32)
    out_ref[0] = r[:, :HALF]
    out_ref[1] = r[:, HALF:]


def _xf(x, lin1_t):
    return pl.pallas_call(
        _xf_body,
        grid=(N // BN,),
        in_specs=[
            pl.BlockSpec((BN, H), lambda i: (i, 0)),
            pl.BlockSpec((H, H), lambda i: (0, 0)),
        ],
        out_specs=pl.BlockSpec((2, BN, HALF), lambda i: (0, i, 0)),
        out_shape=jax.ShapeDtypeStruct((2, N, HALF), jnp.float32),
    )(x, lin1_t)


BE = 1280


def _ssp(v):
    return jax.nn.softplus(v) - jnp.log(2.0)


def _filt_body(eat_ref, el_ref, w1_ref, b1_ref, w2_ref, b2_ref, out_ref):
    ht = jnp.dot(w1_ref[...], eat_ref[...], preferred_element_type=jnp.float32)
    ht = _ssp(ht + b1_ref[...])
    wt = jnp.dot(w2_ref[...], ht, preferred_element_type=jnp.float32) + b2_ref[...]
    el = el_ref[...]
    c = 0.5 * (jnp.cos(el * (jnp.pi / CUTOFF)) + 1.0)
    c = c * (el <= CUTOFF).astype(jnp.float32) * (el >= 0.0).astype(jnp.float32)
    wt = wt * c
    w = wt.T
    out_ref[0] = w[:, :HALF]
    out_ref[1] = w[:, HALF:]


def _filt(edge_attr_t, el2d, w1, b1, w2, b2, eg):
    return pl.pallas_call(
        _filt_body,
        grid=(eg // BE,),
        in_specs=[
            pl.BlockSpec((G, BE), lambda i: (0, i)),
            pl.BlockSpec((1, BE), lambda i: (0, i)),
            pl.BlockSpec((H, G), lambda i: (0, 0)),
            pl.BlockSpec((H, 1), lambda i: (0, 0)),
            pl.BlockSpec((H, H), lambda i: (0, 0)),
            pl.BlockSpec((H, 1), lambda i: (0, 0)),
        ],
        out_specs=pl.BlockSpec((2, BE, HALF), lambda i: (0, i, 0)),
        out_shape=jax.ShapeDtypeStruct((2, eg, HALF), jnp.float32),
    )(edge_attr_t, el2d, w1, b1, w2, b2)


B = 80
EG0 = 81920
EG1 = E - EG0
SPS = 632
NP = 16 * SPS
PK = HALF // 2


def _sc_aggregate(xf2, w2, src2, dst, eg):
    epsg = eg // 16
    nch = epsg // B

    def _sc_body(xf2, w2, src2, dst, agg2, sidxs, didx0, didx1, xr0, xr1, wr,
                 accsh, semg0, semg1, semw, semd0, semd1, semsc0, semsc1):
        c = lax.axis_index("c")
        s = lax.axis_index("s")

        pltpu.sync_copy(src2.at[pl.ds(c * eg + s * epsg, epsg)], sidxs)

        zero16 = jnp.zeros((16,), jnp.float32)

        def zrow(i, carry):
            for j in range(HALF // 16):
                xr0[i, pl.ds(j * 16, 16)] = zero16
            return carry

        lax.fori_loop(0, B, zrow, 0)
        full, rem = divmod(SPS, B)
        for t in range(full):
            pltpu.sync_copy(xr0, accsh.at[pl.ds(s * SPS + t * B, B)])
        if rem:
            pltpu.sync_copy(xr0.at[pl.ds(0, rem)],
                            accsh.at[pl.ds(s * SPS + full * B, rem)])
        plsc.subcore_barrier()

        ebase = s * epsg
        gbase = c * eg + ebase

        pltpu.async_copy(dst.at[pl.ds(ebase, B)], didx0, semd0)
        pltpu.async_copy(w2.at[pl.ds(gbase, B)], wr, semw)
        pltpu.async_copy(xf2.at[sidxs.at[pl.ds(0, B)]], xr0, semg0)

        def chunk(k, carry):
            def run(xr, semg, di, semd, semsc, xr_o, semg_o, di_o, semd_o,
                    semsc_o):
                @pl.when(k >= 1)
                def _drain_sc():
                    pltpu.make_async_copy(xr_o, accsh.at[di_o],
                                          semsc_o).wait()

                @pl.when(k + 1 < nch)
                def _issue():
                    pltpu.async_copy(
                        xf2.at[sidxs.at[pl.ds((k + 1) * B, B)]], xr_o,
                        semg_o)
                    pltpu.async_copy(
                        dst.at[pl.ds(ebase + (k + 1) * B, B)], di_o, semd_o)

                pltpu.make_async_copy(xf2.at[pl.ds(0, B)], xr, semg).wait()
                pltpu.make_async_copy(w2.at[pl.ds(0, B)], wr, semw).wait()

                @plsc.parallel_loop(0, B, 1, unroll=4)
                def mul(i):
                    for j in range(HALF // 16):
                        sl = pl.ds(j * 16, 16)
                        xr[i, sl] = xr[i, sl] * wr[i, sl]

                @pl.when(k + 1 < nch)
                def _issue_w():
                    pltpu.async_copy(
                        w2.at[pl.ds(gbase + (k + 1) * B, B)], wr, semw)

                pltpu.make_async_copy(dst.at[pl.ds(0, B)], di, semd).wait()
                pltpu.async_copy(xr, accsh.at[di], semsc, add=True)

            @pl.when(k % 2 == 0)
            def _even():
                run(xr0, semg0, didx0, semd0, semsc0,
                    xr1, semg1, didx1, semd1, semsc1)

            @pl.when(k % 2 == 1)
            def _odd():
                run(xr1, semg1, didx1, semd1, semsc1,
                    xr0, semg0, didx0, semd0, semsc0)

            return carry

        lax.fori_loop(0, nch, chunk, 0)
        if (nch - 1) % 2 == 0:
            pltpu.make_async_copy(xr0, accsh.at[didx0], semsc0).wait()
        else:
            pltpu.make_async_copy(xr1, accsh.at[didx1], semsc1).wait()
        plsc.subcore_barrier()

        pltpu.sync_copy(accsh.at[pl.ds(s * SPS, SPS)],
                        agg2.at[pl.ds(c * NP + s * SPS, SPS)])

    mesh = plsc.VectorSubcoreMesh(core_axis_name="c", subcore_axis_name="s",
                                  num_cores=2, num_subcores=16)
    return pl.kernel(
        _sc_body,
        out_type=jax.ShapeDtypeStruct((2 * NP, HALF), jnp.float32),
        mesh=mesh,
        scratch_types=[
            pltpu.VMEM((epsg,), jnp.int32),
            pltpu.VMEM((B,), jnp.int32),
            pltpu.VMEM((B,), jnp.int32),
            pltpu.VMEM((B, HALF), jnp.float32),
            pltpu.VMEM((B, HALF), jnp.float32),
            pltpu.VMEM((B, HALF), jnp.float32),
            pltpu.VMEM_SHARED((NP, HALF), jnp.float32),
            pltpu.SemaphoreType.DMA,
            pltpu.SemaphoreType.DMA,
            pltpu.SemaphoreType.DMA,
            pltpu.SemaphoreType.DMA,
            pltpu.SemaphoreType.DMA,
            pltpu.SemaphoreType.DMA,
            pltpu.SemaphoreType.DMA,
        ],
    )(xf2, w2, src2, dst)


def _out_body(g0_ref, g1_ref, w2_ref, b2_ref, w_ref, b_ref, out_ref):
    a0 = g0_ref[0] + g1_ref[0]
    a1 = g0_ref[1] + g1_ref[1]
    r = (jnp.dot(a0, w2_ref[:HALF], preferred_element_type=jnp.float32)
         + jnp.dot(a1, w2_ref[HALF:], preferred_element_type=jnp.float32)
         + b2_ref[...])
    r = _ssp(r)
    out_ref[...] = jnp.dot(r, w_ref[...], preferred_element_type=jnp.float32) + b_ref[...]


def _out_mlp(agg0, agg1, lin2_t, lin2_b, lin_t, lin_b):
    return pl.pallas_call(
        _out_body,
        grid=(N // BN,),
        in_specs=[
            pl.BlockSpec((2, BN, HALF), lambda i: (0, i, 0)),
            pl.BlockSpec((2, BN, HALF), lambda i: (0, i, 0)),
            pl.BlockSpec((H, H), lambda i: (0, 0)),
            pl.BlockSpec((1, H), lambda i: (0, 0)),
            pl.BlockSpec((H, H), lambda i: (0, 0)),
            pl.BlockSpec((1, H), lambda i: (0, 0)),
        ],
        out_specs=pl.BlockSpec((BN, H), lambda i: (i, 0)),
        out_shape=jax.ShapeDtypeStruct((N, H), jnp.float32),
    )(agg0, agg1, lin2_t, lin2_b, lin_t, lin_b)


_PERM = np.array(
    [32 * (c // 32) + 2 * (c % 32) if (c % 32) < 16
     else 32 * (c // 32) + 2 * ((c % 32) - 16) + 1
     for c in range(HALF)], dtype=np.int32)
_PERM_FULL = np.concatenate([_PERM, HALF + _PERM])


def kernel(x, edge_index, edge_length, edge_attr, lin1_w, mlp_w1, mlp_b1,
           mlp_w2, mlp_b2, lin2_w, lin2_b, lin_w, lin_b):
    src = edge_index[0].astype(jnp.int32)
    dst = edge_index[1].astype(jnp.int32)
    eat = edge_attr.T

    xf2 = _xf(x, lin1_w.T).reshape(2 * N, HALF)

    aggs = []
    for lo, eg in ((0, EG0), (EG0, EG1)):
        w2_g = _filt(eat[:, lo:lo + eg],
                     lax.dynamic_slice_in_dim(edge_length, lo, eg).reshape(1, eg),
                     mlp_w1, mlp_b1.reshape(H, 1), mlp_w2,
                     mlp_b2.reshape(H, 1), eg).reshape(2 * eg, HALF)
        src_g = lax.dynamic_slice_in_dim(src, lo, eg)
        dst_g = lax.dynamic_slice_in_dim(dst, lo, eg)
        src2_g = jnp.concatenate([src_g, src_g + N])
        aggs.append(_sc_aggregate(xf2, w2_g, src2_g, dst_g,
                                  eg).reshape(2, NP, HALF))
    out = _out_mlp(aggs[0], aggs[1], lin2_w.T, lin2_b.reshape(1, H),
                   lin_w.T, lin_b.reshape(1, H))
    return out


# speedup vs baseline: 3.6858x; 1.0243x over previous
import jax
import jax.numpy as jnp
import numpy as np
from jax import lax
from jax.experimental import pallas as pl
from jax.experimental.pallas import tpu as pltpu
from jax.experimental.pallas import tpu_sc as plsc

N = 10000
E = 160000
H = 256
HALF = 128
G = 64
CUTOFF = 10.0


BN = 2000


def _xf_body(x_ref, w_ref, out_ref):
    r = jnp.dot(x_ref[...], w_ref[...], preferred_element_type=jnp.float32)
    out_ref[0] = r[:, :HALF]
    out_ref[1] = r[:, HALF:]


def _xf(x, lin1_t):
    return pl.pallas_call(
        _xf_body,
        grid=(N // BN,),
        in_specs=[
            pl.BlockSpec((BN, H), lambda i: (i, 0)),
            pl.BlockSpec((H, H), lambda i: (0, 0)),
        ],
        out_specs=pl.BlockSpec((2, BN, HALF), lambda i: (0, i, 0)),
        out_shape=jax.ShapeDtypeStruct((2, N, HALF), jnp.float32),
    )(x, lin1_t)


BE = 1280


def _ssp(v):
    return jax.nn.softplus(v) - jnp.log(2.0)


def _filt_body(eat_ref, el_ref, w1_ref, b1_ref, w2_ref, b2_ref, out_ref):
    ht = jnp.dot(w1_ref[...], eat_ref[...], preferred_element_type=jnp.float32)
    ht = _ssp(ht + b1_ref[...])
    wt = jnp.dot(w2_ref[...], ht, preferred_element_type=jnp.float32) + b2_ref[...]
    el = el_ref[...]
    c = 0.5 * (jnp.cos(el * (jnp.pi / CUTOFF)) + 1.0)
    c = c * (el <= CUTOFF).astype(jnp.float32) * (el >= 0.0).astype(jnp.float32)
    wt = wt * c
    w = wt.T
    out_ref[0] = w[:, :HALF]
    out_ref[1] = w[:, HALF:]


def _filt(edge_attr_t, el2d, w1, b1, w2, b2, eg):
    return pl.pallas_call(
        _filt_body,
        grid=(eg // BE,),
        in_specs=[
            pl.BlockSpec((G, BE), lambda i: (0, i)),
            pl.BlockSpec((1, BE), lambda i: (0, i)),
            pl.BlockSpec((H, G), lambda i: (0, 0)),
            pl.BlockSpec((H, 1), lambda i: (0, 0)),
            pl.BlockSpec((H, H), lambda i: (0, 0)),
            pl.BlockSpec((H, 1), lambda i: (0, 0)),
        ],
        out_specs=pl.BlockSpec((2, BE, HALF), lambda i: (0, i, 0)),
        out_shape=jax.ShapeDtypeStruct((2, eg, HALF), jnp.float32),
    )(edge_attr_t, el2d, w1, b1, w2, b2)


B = 80
EGROUPS = (40960, 39680, 39680, 39680)
SPS = 632
NP = 16 * SPS
PK = HALF // 2


def _sc_aggregate(xf2, w2, src2, dst, eg):
    epsg = eg // 16
    nch = epsg // B

    def _sc_body(xf2, w2, src2, dst, agg2, sidxs, didx0, didx1, xr0, xr1, wr,
                 accsh, semg0, semg1, semw, semd0, semd1, semsc0, semsc1):
        c = lax.axis_index("c")
        s = lax.axis_index("s")

        pltpu.sync_copy(src2.at[pl.ds(c * eg + s * epsg, epsg)], sidxs)

        zero16 = jnp.zeros((16,), jnp.float32)

        def zrow(i, carry):
            for j in range(HALF // 16):
                xr0[i, pl.ds(j * 16, 16)] = zero16
            return carry

        lax.fori_loop(0, B, zrow, 0)
        full, rem = divmod(SPS, B)
        for t in range(full):
            pltpu.sync_copy(xr0, accsh.at[pl.ds(s * SPS + t * B, B)])
        if rem:
            pltpu.sync_copy(xr0.at[pl.ds(0, rem)],
                            accsh.at[pl.ds(s * SPS + full * B, rem)])
        plsc.subcore_barrier()

        ebase = s * epsg
        gbase = c * eg + ebase

        pltpu.async_copy(dst.at[pl.ds(ebase, B)], didx0, semd0)
        pltpu.async_copy(w2.at[pl.ds(gbase, B)], wr, semw)
        pltpu.async_copy(xf2.at[sidxs.at[pl.ds(0, B)]], xr0, semg0)

        def chunk(k, carry):
            def run(xr, semg, di, semd, semsc, xr_o, semg_o, di_o, semd_o,
                    semsc_o):
                @pl.when(k >= 1)
                def _drain_sc():
                    pltpu.make_async_copy(xr_o, accsh.at[di_o],
                                          semsc_o).wait()

                @pl.when(k + 1 < nch)
                def _issue():
                    pltpu.async_copy(
                        xf2.at[sidxs.at[pl.ds((k + 1) * B, B)]], xr_o,
                        semg_o)
                    pltpu.async_copy(
                        dst.at[pl.ds(ebase + (k + 1) * B, B)], di_o, semd_o)

                pltpu.make_async_copy(xf2.at[pl.ds(0, B)], xr, semg).wait()
                pltpu.make_async_copy(w2.at[pl.ds(0, B)], wr, semw).wait()

                @plsc.parallel_loop(0, B, 1, unroll=4)
                def mul(i):
                    for j in range(HALF // 16):
                        sl = pl.ds(j * 16, 16)
                        xr[i, sl] = xr[i, sl] * wr[i, sl]

                @pl.when(k + 1 < nch)
                def _issue_w():
                    pltpu.async_copy(
                        w2.at[pl.ds(gbase + (k + 1) * B, B)], wr, semw)

                pltpu.make_async_copy(dst.at[pl.ds(0, B)], di, semd).wait()
                pltpu.async_copy(xr, accsh.at[di], semsc, add=True)

            @pl.when(k % 2 == 0)
            def _even():
                run(xr0, semg0, didx0, semd0, semsc0,
                    xr1, semg1, didx1, semd1, semsc1)

            @pl.when(k % 2 == 1)
            def _odd():
                run(xr1, semg1, didx1, semd1, semsc1,
                    xr0, semg0, didx0, semd0, semsc0)

            return carry

        lax.fori_loop(0, nch, chunk, 0)
        if (nch - 1) % 2 == 0:
            pltpu.make_async_copy(xr0, accsh.at[didx0], semsc0).wait()
        else:
            pltpu.make_async_copy(xr1, accsh.at[didx1], semsc1).wait()
        plsc.subcore_barrier()

        pltpu.sync_copy(accsh.at[pl.ds(s * SPS, SPS)],
                        agg2.at[pl.ds(c * NP + s * SPS, SPS)])

    mesh = plsc.VectorSubcoreMesh(core_axis_name="c", subcore_axis_name="s",
                                  num_cores=2, num_subcores=16)
    return pl.kernel(
        _sc_body,
        out_type=jax.ShapeDtypeStruct((2 * NP, HALF), jnp.float32),
        mesh=mesh,
        scratch_types=[
            pltpu.VMEM((epsg,), jnp.int32),
            pltpu.VMEM((B,), jnp.int32),
            pltpu.VMEM((B,), jnp.int32),
            pltpu.VMEM((B, HALF), jnp.float32),
            pltpu.VMEM((B, HALF), jnp.float32),
            pltpu.VMEM((B, HALF), jnp.float32),
            pltpu.VMEM_SHARED((NP, HALF), jnp.float32),
            pltpu.SemaphoreType.DMA,
            pltpu.SemaphoreType.DMA,
            pltpu.SemaphoreType.DMA,
            pltpu.SemaphoreType.DMA,
            pltpu.SemaphoreType.DMA,
            pltpu.SemaphoreType.DMA,
            pltpu.SemaphoreType.DMA,
        ],
    )(xf2, w2, src2, dst)


def _out_body(*refs):
    g_refs = refs[:len(EGROUPS)]
    w2_ref, b2_ref, w_ref, b_ref, out_ref = refs[len(EGROUPS):]
    a0 = g_refs[0][0]
    a1 = g_refs[0][1]
    for g_ref in g_refs[1:]:
        a0 = a0 + g_ref[0]
        a1 = a1 + g_ref[1]
    r = (jnp.dot(a0, w2_ref[:HALF], preferred_element_type=jnp.float32)
         + jnp.dot(a1, w2_ref[HALF:], preferred_element_type=jnp.float32)
         + b2_ref[...])
    r = _ssp(r)
    out_ref[...] = jnp.dot(r, w_ref[...], preferred_element_type=jnp.float32) + b_ref[...]


def _out_mlp(aggs, lin2_t, lin2_b, lin_t, lin_b):
    return pl.pallas_call(
        _out_body,
        grid=(N // BN,),
        in_specs=[
            pl.BlockSpec((2, BN, HALF), lambda i: (0, i, 0))
            for _ in EGROUPS
        ] + [
            pl.BlockSpec((H, H), lambda i: (0, 0)),
            pl.BlockSpec((1, H), lambda i: (0, 0)),
            pl.BlockSpec((H, H), lambda i: (0, 0)),
            pl.BlockSpec((1, H), lambda i: (0, 0)),
        ],
        out_specs=pl.BlockSpec((BN, H), lambda i: (i, 0)),
        out_shape=jax.ShapeDtypeStruct((N, H), jnp.float32),
    )(*aggs, lin2_t, lin2_b, lin_t, lin_b)


_PERM = np.array(
    [32 * (c // 32) + 2 * (c % 32) if (c % 32) < 16
     else 32 * (c // 32) + 2 * ((c % 32) - 16) + 1
     for c in range(HALF)], dtype=np.int32)
_PERM_FULL = np.concatenate([_PERM, HALF + _PERM])


def kernel(x, edge_index, edge_length, edge_attr, lin1_w, mlp_w1, mlp_b1,
           mlp_w2, mlp_b2, lin2_w, lin2_b, lin_w, lin_b):
    src = edge_index[0].astype(jnp.int32)
    dst = edge_index[1].astype(jnp.int32)
    eat = edge_attr.T

    xf2 = _xf(x, lin1_w.T).reshape(2 * N, HALF)

    aggs = []
    los = [0]
    for eg in EGROUPS[:-1]:
        los.append(los[-1] + eg)
    for lo, eg in zip(los, EGROUPS):
        w2_g = _filt(eat[:, lo:lo + eg],
                     lax.dynamic_slice_in_dim(edge_length, lo, eg).reshape(1, eg),
                     mlp_w1, mlp_b1.reshape(H, 1), mlp_w2,
                     mlp_b2.reshape(H, 1), eg).reshape(2 * eg, HALF)
        src_g = lax.dynamic_slice_in_dim(src, lo, eg)
        dst_g = lax.dynamic_slice_in_dim(dst, lo, eg)
        src2_g = jnp.concatenate([src_g, src_g + N])
        aggs.append(_sc_aggregate(xf2, w2_g, src2_g, dst_g,
                                  eg).reshape(2, NP, HALF))
    out = _out_mlp(aggs, lin2_w.T, lin2_b.reshape(1, H),
                   lin_w.T, lin_b.reshape(1, H))
    return out
